# baseline (speedup 1.0000x reference)
import functools

import jax
import jax.numpy as jnp
from jax import lax
from jax.experimental import pallas as pl
from jax.experimental.pallas import tpu as pltpu
from jax.experimental.pallas import tpu_sc as plsc

_NUM_CORES = 2
_NUM_SUBCORES = 16
_NUM_WORKERS = _NUM_CORES * _NUM_SUBCORES


def kernel(tokens, pe):
    B, L = tokens.shape
    _, D = pe.shape

    rows_per_worker = B // _NUM_WORKERS
    RB = 4
    n_dma = rows_per_worker // RB

    mesh = plsc.VectorSubcoreMesh(core_axis_name="c", subcore_axis_name="s")

    RB_S = 2
    n_spm = 24
    spm_rows = n_spm * RB_S
    n_dma = (rows_per_worker - spm_rows) // RB

    @functools.partial(
        pl.kernel,
        out_type=jax.ShapeDtypeStruct((B * L, D), jnp.float32),
        mesh=mesh,
        scratch_types=[
            pltpu.VMEM((RB * L, D), jnp.float32),
            pltpu.VMEM_SHARED((_NUM_SUBCORES // 2 * RB_S * L, D), jnp.float32),
            pltpu.SemaphoreType.DMA,
        ],
    )
    def pe_broadcast(pe_hbm, out_hbm, rep_v, rep_s, sem):
        cid = lax.axis_index("c")
        sid = lax.axis_index("s")
        wid = sid * _NUM_CORES + cid
        base = wid * rows_per_worker * L
        sbase = (sid // 2) * RB_S * L
        for j in range(RB):
            pltpu.make_async_copy(
                pe_hbm.at[pl.ds(0, L)], rep_v.at[pl.ds(j * L, L)], sem
            ).start()
        for j in range(RB_S):
            pltpu.make_async_copy(
                pe_hbm.at[pl.ds(0, L)], rep_s.at[pl.ds(sbase + j * L, L)], sem
            ).start()
        for j in range(RB):
            pltpu.make_async_copy(
                pe_hbm.at[pl.ds(0, L)], rep_v.at[pl.ds(j * L, L)], sem
            ).wait()
        for j in range(RB_S):
            pltpu.make_async_copy(
                pe_hbm.at[pl.ds(0, L)], rep_s.at[pl.ds(sbase + j * L, L)], sem
            ).wait()
        tbase = base + spm_rows * L
        for i in range(n_spm):
            pltpu.make_async_copy(
                rep_s.at[pl.ds(sbase, RB_S * L)],
                out_hbm.at[pl.ds(base + i * RB_S * L, RB_S * L)],
                sem,
            ).start()
        for i in range(n_dma):
            pltpu.make_async_copy(
                rep_v, out_hbm.at[pl.ds(tbase + i * RB * L, RB * L)], sem
            ).start()
        for i in range(n_spm):
            pltpu.make_async_copy(
                rep_s.at[pl.ds(sbase, RB_S * L)],
                out_hbm.at[pl.ds(base + i * RB_S * L, RB_S * L)],
                sem,
            ).wait()
        for i in range(n_dma):
            pltpu.make_async_copy(
                rep_v, out_hbm.at[pl.ds(tbase + i * RB * L, RB * L)], sem
            ).wait()

    out = pe_broadcast(pe)
    return out.reshape(B, L, D)


# speedup vs baseline: 1.1169x; 1.1169x over previous
import functools

import jax
import jax.numpy as jnp
from jax import lax
from jax.experimental import pallas as pl
from jax.experimental.pallas import tpu as pltpu
from jax.experimental.pallas import tpu_sc as plsc

_NUM_CORES = 2
_NUM_SUBCORES = 16
_NUM_WORKERS = _NUM_CORES * _NUM_SUBCORES


def kernel(tokens, pe):
    B, L = tokens.shape
    _, D = pe.shape

    rows_per_worker = B // _NUM_WORKERS
    RB = 2
    n_dma = rows_per_worker // RB

    mesh = plsc.VectorSubcoreMesh(core_axis_name="c", subcore_axis_name="s")

    @functools.partial(
        pl.kernel,
        out_type=jax.ShapeDtypeStruct((B * L, D), jnp.float32),
        mesh=mesh,
        scratch_types=[
            pltpu.VMEM((RB * L, D), jnp.float32),
            pltpu.SemaphoreType.DMA,
        ],
    )
    def pe_broadcast(pe_hbm, out_hbm, rep_v, sem):
        wid = lax.axis_index("s") * _NUM_CORES + lax.axis_index("c")
        base = wid * rows_per_worker * L
        for j in range(RB):
            pltpu.make_async_copy(
                pe_hbm.at[pl.ds(0, L)], rep_v.at[pl.ds(j * L, L)], sem
            ).start()
        for j in range(RB):
            pltpu.make_async_copy(
                pe_hbm.at[pl.ds(0, L)], rep_v.at[pl.ds(j * L, L)], sem
            ).wait()
        for i in range(n_dma):
            pltpu.make_async_copy(
                rep_v, out_hbm.at[pl.ds(base + i * RB * L, RB * L)], sem
            ).start()
        for i in range(n_dma):
            pltpu.make_async_copy(
                rep_v, out_hbm.at[pl.ds(base + i * RB * L, RB * L)], sem
            ).wait()

    out = pe_broadcast(pe)
    return out.reshape(B, L, D)


# speedup vs baseline: 1.1207x; 1.0034x over previous
import functools

import jax
import jax.numpy as jnp
from jax import lax
from jax.experimental import pallas as pl
from jax.experimental.pallas import tpu as pltpu
from jax.experimental.pallas import tpu_sc as plsc

_NUM_CORES = 2
_NUM_SUBCORES = 16
_NUM_WORKERS = _NUM_CORES * _NUM_SUBCORES


def kernel(tokens, pe):
    B, L = tokens.shape
    _, D = pe.shape

    rows_per_worker = B // _NUM_WORKERS
    RB = 1
    n_dma = rows_per_worker // RB

    mesh = plsc.VectorSubcoreMesh(core_axis_name="c", subcore_axis_name="s")

    @functools.partial(
        pl.kernel,
        out_type=jax.ShapeDtypeStruct((B * L, D), jnp.float32),
        mesh=mesh,
        scratch_types=[
            pltpu.VMEM((RB * L, D), jnp.float32),
            pltpu.SemaphoreType.DMA,
        ],
    )
    def pe_broadcast(pe_hbm, out_hbm, rep_v, sem):
        wid = lax.axis_index("s") * _NUM_CORES + lax.axis_index("c")
        base = wid * rows_per_worker * L
        for j in range(RB):
            pltpu.make_async_copy(
                pe_hbm.at[pl.ds(0, L)], rep_v.at[pl.ds(j * L, L)], sem
            ).start()
        for j in range(RB):
            pltpu.make_async_copy(
                pe_hbm.at[pl.ds(0, L)], rep_v.at[pl.ds(j * L, L)], sem
            ).wait()
        for i in range(n_dma):
            pltpu.make_async_copy(
                rep_v, out_hbm.at[pl.ds(base + i * RB * L, RB * L)], sem
            ).start()
        for i in range(n_dma):
            pltpu.make_async_copy(
                rep_v, out_hbm.at[pl.ds(base + i * RB * L, RB * L)], sem
            ).wait()

    out = pe_broadcast(pe)
    return out.reshape(B, L, D)


# speedup vs baseline: 1.1404x; 1.0176x over previous
import functools

import jax
import jax.numpy as jnp
from jax import lax
from jax.experimental import pallas as pl
from jax.experimental.pallas import tpu as pltpu
from jax.experimental.pallas import tpu_sc as plsc

_NUM_CORES = 2
_NUM_SUBCORES = 16


def kernel(tokens, pe):
    B, L = tokens.shape
    _, D = pe.shape

    rows_per_pair = B // _NUM_SUBCORES
    r_lo = 123
    r_hi = rows_per_pair - r_lo

    mesh = plsc.VectorSubcoreMesh(core_axis_name="c", subcore_axis_name="s")

    @functools.partial(
        pl.kernel,
        out_type=jax.ShapeDtypeStruct((B * L, D), jnp.float32),
        mesh=mesh,
        scratch_types=[
            pltpu.VMEM((L, D), jnp.float32),
            pltpu.SemaphoreType.DMA,
        ],
    )
    def pe_broadcast(pe_hbm, out_hbm, rep_v, sem):
        cid = lax.axis_index("c")
        sid = lax.axis_index("s")
        base = (sid * rows_per_pair + cid * r_lo) * L
        pltpu.sync_copy(pe_hbm.at[pl.ds(0, L)], rep_v)
        for i in range(r_lo):
            pltpu.make_async_copy(
                rep_v, out_hbm.at[pl.ds(base + i * L, L)], sem
            ).start()

        @pl.when(cid == 1)
        def _():
            for i in range(r_lo, r_hi):
                pltpu.make_async_copy(
                    rep_v, out_hbm.at[pl.ds(base + i * L, L)], sem
                ).start()

        for i in range(r_lo):
            pltpu.make_async_copy(
                rep_v, out_hbm.at[pl.ds(base + i * L, L)], sem
            ).wait()

        @pl.when(cid == 1)
        def _():
            for i in range(r_lo, r_hi):
                pltpu.make_async_copy(
                    rep_v, out_hbm.at[pl.ds(base + i * L, L)], sem
                ).wait()

    out = pe_broadcast(pe)
    return out.reshape(B, L, D)
